# Initial kernel scaffold; baseline (speedup 1.0000x reference)
import jax
import jax.numpy as jnp
from jax.experimental import pallas as pl


def kernel(X, edge_index, hs, cs, W, b, theta, conv_bias):
    raise NotImplementedError("write your pallas kernel here")


# speedup vs baseline: 23.3883x; 23.3883x over previous
import functools

import jax
import jax.numpy as jnp
from jax import lax
from jax.experimental import pallas as pl
from jax.experimental.pallas import tpu as pltpu
from jax.experimental.pallas import tpu_sc as plsc

N = 10000
E = 320000
D = 128
L = 2

NTILES = 16
EPT = E // NTILES
CHUNK = 80
BLKE = 4000
NBLK = EPT // BLKE
BCH = BLKE // CHUNK
NPAD = 10240
NPT = NPAD // NTILES
TRASH = N


def _floop(lo, hi, body):
    lax.fori_loop(lo, hi, lambda i, c: (body(i), c)[1], 0, unroll=False)


def _rsqrt16(x):
    i = plsc.bitcast(x, jnp.int32)
    i = jnp.full((16,), 0x5F3759DF, jnp.int32) - (i >> 1)
    y = plsc.bitcast(i, jnp.float32)
    for _ in range(3):
        y = y * (1.5 - 0.5 * x * y * y)
    return jnp.where(x > 0.0, y, jnp.full((16,), 0.0, jnp.float32))


def _sc_body(row_hbm, col_hbm, hs_hbm, tx_hbm, hsc_hbm,
             erow_v, ecol_v, deg_v, dinv_v, hbuf_v, zbuf_v, obuf_v,
             rows2_v, gidx2_v, colm2_v, deg_sh, acc_sh,
             sem0, sem1):
    c = lax.axis_index("c")
    s = lax.axis_index("s")
    cN = c * NPAD
    ebase = s * EPT
    nbase = s * NPT

    zero16 = jnp.full((16,), 0.0, jnp.float32)
    one16 = jnp.full((16,), 1.0, jnp.float32)
    trash16 = jnp.full((16,), TRASH, jnp.int32)
    iota16 = lax.broadcasted_iota(jnp.int32, (16,), 0)

    def _zb_row(i):
        for k in range(8):
            zbuf_v[i, pl.ds(k * 16, 16)] = zero16
    _floop(0, 8, _zb_row)

    def _zero_deg(i):
        for k in range(8):
            deg_v[i, pl.ds(k * 16, 16)] = zero16
    _floop(0, NPAD // D, _zero_deg)

    @pl.when(s == 0)
    def _():
        def _zdsh(q):
            pltpu.sync_copy(zbuf_v, deg_sh.at[pl.ds(q * 8, 8)])
        _floop(0, (NPAD // D) // 8, _zdsh)

    def _ablock(bi):
        eb = ebase + bi * BLKE
        pltpu.sync_copy(row_hbm.at[pl.ds(eb, BLKE)], erow_v)
        pltpu.sync_copy(col_hbm.at[pl.ds(eb, BLKE)], ecol_v)

        def _deg_step(k):
            r = erow_v[pl.ds(k * 16, 16)]
            cc = ecol_v[pl.ds(k * 16, 16)]
            rm = jnp.where(r == cc, trash16, r)
            plsc.addupdate_scatter(deg_v, [rm >> 7, rm & 127], one16)
        _floop(0, BLKE // 16, _deg_step)
    _floop(0, NBLK, _ablock)

    for k in range(5):
        gidx2_v[0, pl.ds(k * 16, 16)] = iota16 + (k * 16)
    plsc.subcore_barrier()
    pltpu.sync_copy(deg_v, deg_sh.at[gidx2_v.at[0]], add=True)
    plsc.subcore_barrier()

    pltpu.sync_copy(deg_sh.at[pl.ds(nbase // D, NPT // D)],
                    deg_v.at[pl.ds(0, NPT // D)])

    def _dinv_step(i):
        dv = _rsqrt16(deg_v[i // 8, pl.ds((i % 8) * 16, 16)])
        dinv_v[pl.ds(i * 16, 16)] = dv
    _floop(0, NPT // 16, _dinv_step)

    HROWS = 32

    def _hs_chunk(q):
        r0 = q * HROWS
        pltpu.sync_copy(hs_hbm.at[pl.ds(cN + nbase + r0, HROWS)], hbuf_v)
        for j in range(HROWS // 16):
            dv = -dinv_v[pl.ds(r0 + j * 16, 16)]
            for r in range(16):
                scv = jnp.full((16,), dv[r], jnp.float32)
                for k in range(8):
                    hbuf_v[j * 16 + r, pl.ds(k * 16, 16)] = (
                        hbuf_v[j * 16 + r, pl.ds(k * 16, 16)] * scv)
        pltpu.sync_copy(hbuf_v, hsc_hbm.at[pl.ds(cN + nbase + r0, HROWS)])
    _floop(0, NPT // HROWS, _hs_chunk)

    def _zacc(q):
        pltpu.sync_copy(zbuf_v, acc_sh.at[pl.ds(nbase + q * 8, 8)])
    _floop(0, NPT // 8, _zacc)

    plsc.subcore_barrier()

    sems = (sem0, sem1)

    def _stage(ch, b):
        base = ch * CHUNK
        for k in range(CHUNK // 16):
            r = erow_v[pl.ds(base + k * 16, 16)]
            cc = ecol_v[pl.ds(base + k * 16, 16)]
            gidx2_v[b, pl.ds(k * 16, 16)] = r + cN
            colm2_v[b, pl.ds(k * 16, 16)] = jnp.where(r == cc, trash16, cc)

    def _start(b):
        pltpu.async_copy(hsc_hbm.at[gidx2_v.at[b]], rows2_v.at[b], sems[b])

    def _finish(b):
        pltpu.make_async_copy(hsc_hbm.at[gidx2_v.at[b]], rows2_v.at[b],
                              sems[b]).wait()
        pltpu.sync_copy(rows2_v.at[b], acc_sh.at[colm2_v.at[b]], add=True)

    def _cblock(bi):
        eb = ebase + bi * BLKE
        pltpu.sync_copy(row_hbm.at[pl.ds(eb, BLKE)], erow_v)
        pltpu.sync_copy(col_hbm.at[pl.ds(eb, BLKE)], ecol_v)
        _stage(0, 0)
        _start(0)

        def _pc_iter(t):
            for b in range(2):
                ch = t * 2 + b

                @pl.when(ch + 1 < BCH)
                def _():
                    _stage(ch + 1, 1 - b)
                    _start(1 - b)

                _finish(b)
        _floop(0, BCH // 2, _pc_iter)
    _floop(0, NBLK, _cblock)

    plsc.subcore_barrier()

    ORows = 16

    def _out_chunk(q):
        r0 = q * ORows

        @pl.when(nbase + r0 < N)
        def _():
            pltpu.sync_copy(acc_sh.at[pl.ds(nbase + r0, ORows)], obuf_v)
            dv = dinv_v[pl.ds(r0, 16)]
            for r in range(16):
                scv = jnp.full((16,), dv[r], jnp.float32)
                for k in range(8):
                    obuf_v[r, pl.ds(k * 16, 16)] = (
                        obuf_v[r, pl.ds(k * 16, 16)] * scv)
            pltpu.sync_copy(obuf_v, tx_hbm.at[pl.ds(c * N + nbase + r0, ORows)])
    _floop(0, NPT // ORows, _out_chunk)


@jax.jit
def _sc_scatter(row, col, hs_cat):
    mesh = plsc.VectorSubcoreMesh(core_axis_name="c", subcore_axis_name="s")
    f = pl.kernel(
        _sc_body,
        out_type=(
            jax.ShapeDtypeStruct((L * N, D), jnp.float32),
            jax.ShapeDtypeStruct((L * NPAD, D), jnp.float32),
        ),
        mesh=mesh,
        scratch_types=[
            pltpu.VMEM((BLKE,), jnp.int32),
            pltpu.VMEM((BLKE,), jnp.int32),
            pltpu.VMEM((NPAD // D, D), jnp.float32),
            pltpu.VMEM((NPT,), jnp.float32),
            pltpu.VMEM((32, D), jnp.float32),
            pltpu.VMEM((8, D), jnp.float32),
            pltpu.VMEM((16, D), jnp.float32),
            pltpu.VMEM((2, CHUNK, D), jnp.float32),
            pltpu.VMEM((2, CHUNK), jnp.int32),
            pltpu.VMEM((2, CHUNK), jnp.int32),
            pltpu.VMEM_SHARED((NPAD // D, D), jnp.float32),
            pltpu.VMEM_SHARED((NPAD, D), jnp.float32),
            pltpu.SemaphoreType.DMA,
            pltpu.SemaphoreType.DMA,
        ],
        compiler_params=pltpu.CompilerParams(needs_layout_passes=False),
    )
    tx_cat, _ = f(row, col, hs_cat)
    return tx_cat


def _tc_body(x_ref, hs_ref, cs_ref, tx_ref, w_ref, t0_ref, t1_ref, bias_ref,
             oh_ref, oc_ref):
    x = x_ref[...]
    for l in range(L):
        a = (jnp.dot(x, w_ref[l], preferred_element_type=jnp.float32)
             + jnp.dot(hs_ref[l], t0_ref[l], preferred_element_type=jnp.float32)
             + jnp.dot(tx_ref[l], t1_ref[l], preferred_element_type=jnp.float32)
             + bias_ref[l])
        gi = jax.nn.sigmoid(a[:, 0:D])
        gf = jax.nn.sigmoid(a[:, D:2 * D])
        gt = jnp.tanh(a[:, 2 * D:3 * D])
        go = jax.nn.sigmoid(a[:, 3 * D:4 * D])
        cn = gf * cs_ref[l] + gi * gt
        hn = go * jnp.tanh(cn)
        oh_ref[l] = hn
        oc_ref[l] = cn
        x = hn


@jax.jit
def _tc_dense(X, hs, cs, tx, w_cat, t0_cat, t1_cat, bias_cat):
    BLK = 1000
    grid = (N // BLK,)
    f = pl.pallas_call(
        _tc_body,
        grid=grid,
        in_specs=[
            pl.BlockSpec((BLK, D), lambda i: (i, 0)),
            pl.BlockSpec((L, BLK, D), lambda i: (0, i, 0)),
            pl.BlockSpec((L, BLK, D), lambda i: (0, i, 0)),
            pl.BlockSpec((L, BLK, D), lambda i: (0, i, 0)),
            pl.BlockSpec((L, D, 4 * D), lambda i: (0, 0, 0)),
            pl.BlockSpec((L, D, 4 * D), lambda i: (0, 0, 0)),
            pl.BlockSpec((L, D, 4 * D), lambda i: (0, 0, 0)),
            pl.BlockSpec((L, 1, 4 * D), lambda i: (0, 0, 0)),
        ],
        out_specs=[
            pl.BlockSpec((L, BLK, D), lambda i: (0, i, 0)),
            pl.BlockSpec((L, BLK, D), lambda i: (0, i, 0)),
        ],
        out_shape=[
            jax.ShapeDtypeStruct((L, N, D), jnp.float32),
            jax.ShapeDtypeStruct((L, N, D), jnp.float32),
        ],
    )
    return f(X, hs, cs, tx, w_cat, t0_cat, t1_cat, bias_cat)


def kernel(X, edge_index, hs, cs, W, b, theta, conv_bias):
    row = edge_index[0]
    col = edge_index[1]
    hs_pad = jnp.pad(hs, ((0, 0), (0, NPAD - N), (0, 0))).reshape(L * NPAD, D)

    tx_cat = _sc_scatter(row, col, hs_pad)
    tx = tx_cat.reshape(L, N, D)

    w_cat = jnp.transpose(W, (0, 2, 1, 3)).reshape(L, D, 4 * D)
    t0_cat = jnp.transpose(theta[:, :, 0], (0, 2, 1, 3)).reshape(L, D, 4 * D)
    t1_cat = jnp.transpose(theta[:, :, 1], (0, 2, 1, 3)).reshape(L, D, 4 * D)
    bias_cat = (b + conv_bias).reshape(L, 1, 4 * D)

    oh, oc = _tc_dense(X, hs, cs, tx, w_cat, t0_cat, t1_cat, bias_cat)
    return oh, oc, oh[1]
